# Initial kernel scaffold; baseline (speedup 1.0000x reference)
import jax
import jax.numpy as jnp
from jax.experimental import pallas as pl


def kernel(x, edge_index, edge_attr, params):
    raise NotImplementedError("write your pallas kernel here")


# speedup vs baseline: 1.0598x; 1.0598x over previous
import functools

import jax
import jax.numpy as jnp
from jax.experimental import pallas as pl

_N = 50000
_E = 800000
_HEADS = 2


def _mm_body(x_ref, w_ref, o_ref):
    o_ref[...] = jnp.dot(x_ref[...], w_ref[...],
                         preferred_element_type=jnp.float32)


def _mm(x, w, bn=1024):
    n, k = x.shape
    m = w.shape[1]
    return pl.pallas_call(
        _mm_body,
        grid=(n // bn,),
        in_specs=[pl.BlockSpec((bn, k), lambda i: (i, 0)),
                  pl.BlockSpec((k, m), lambda i: (0, 0))],
        out_specs=pl.BlockSpec((bn, m), lambda i: (i, 0)),
        out_shape=jax.ShapeDtypeStruct((n, m), jnp.float32),
    )(x, w)


def _pad_to(a, rows, cols):
    return jnp.pad(a, ((0, rows - a.shape[0]), (0, cols - a.shape[1])))


def kernel(x, edge_index, edge_attr, params):
    src = edge_index[0].astype(jnp.int32)
    dst = edge_index[1].astype(jnp.int32)

    n_pad = 50176
    e_pad = 800768

    we_cols = []
    for p in params:
        h_, c_ = p['att_e'].shape
        we_vec = (p['W_e'].reshape(-1, h_, c_) * p['att_e'][None]).sum(-1)
        we_cols.append(we_vec)
    we_cat = jnp.concatenate(we_cols, axis=1)
    ae_all = _mm(_pad_to(edge_attr, e_pad, 128),
                 _pad_to(we_cat, 128, 128))[:_E, :16]

    h = _pad_to(x, n_pad, 128)
    num_layers = len(params)
    for i, p in enumerate(params):
        heads, c = p['att_src'].shape
        hc = heads * c
        in_d = p['W'].shape[0]
        w_src = (p['W'].reshape(in_d, heads, c) * p['att_src'][None]).sum(-1)
        w_dst = (p['W'].reshape(in_d, heads, c) * p['att_dst'][None]).sum(-1)
        w_cat = jnp.concatenate([p['W'], w_src, w_dst], axis=1)
        m_cols = 256 if hc + 2 * heads > 128 else 128
        out = _mm(h, _pad_to(w_cat, 128, m_cols))
        xl = out[:_N, :hc]
        a_src = out[:_N, hc:hc + heads]
        a_dst = out[:_N, hc + heads:hc + 2 * heads]

        alpha = a_src[src] + a_dst[dst] + ae_all[:, 2 * i:2 * i + 2]
        alpha = jnp.where(alpha > 0, alpha, 0.2 * alpha)
        m = jax.ops.segment_max(alpha, dst, num_segments=_N)
        pvals = jnp.exp(alpha - m[dst])
        denom = jax.ops.segment_sum(pvals, dst, num_segments=_N)
        msg = xl[src].reshape(_E, heads, c) * pvals[..., None]
        acc = jax.ops.segment_sum(msg, dst, num_segments=_N)
        acc = acc / (denom + 1e-16)[..., None]
        if i < num_layers - 1:
            hn = acc.reshape(_N, hc) + p['b']
            hn = jnp.maximum(hn, 0.0)
            h = jnp.pad(hn, ((0, n_pad - _N), (0, 0)))
        else:
            return acc.mean(axis=1) + p['b']


# speedup vs baseline: 24.2920x; 22.9213x over previous
import functools

import jax
import jax.numpy as jnp
from jax import lax
from jax.experimental import pallas as pl
from jax.experimental.pallas import tpu as pltpu
from jax.experimental.pallas import tpu_sc as plsc

_N = 50000
_E = 800000
_EP = 802816
_CHUNK = 128
_NCHUNKS = 196
_PER_W = _CHUNK * _NCHUNKS
_NC = 2
_NS = 16


def _mm_body(x_ref, w_ref, o_ref):
    o_ref[...] = jnp.dot(x_ref[...], w_ref[...],
                         preferred_element_type=jnp.float32)


def _mm(x, w, bn=1024):
    n, k = x.shape
    m = w.shape[1]
    return pl.pallas_call(
        _mm_body,
        grid=(n // bn,),
        in_specs=[pl.BlockSpec((bn, k), lambda i: (i, 0)),
                  pl.BlockSpec((k, m), lambda i: (0, 0))],
        out_specs=pl.BlockSpec((bn, m), lambda i: (i, 0)),
        out_shape=jax.ShapeDtypeStruct((n, m), jnp.float32),
    )(x, w)


def _pad_to(a, rows, cols):
    return jnp.pad(a, ((0, rows - a.shape[0]), (0, cols - a.shape[1])))


def _worker_id():
    return lax.axis_index("s") * _NC + lax.axis_index("c")


def _alpha_kernel():
    mesh = plsc.VectorSubcoreMesh(core_axis_name="c", subcore_axis_name="s")

    @functools.partial(
        pl.kernel, mesh=mesh,
        compiler_params=pltpu.CompilerParams(use_tc_tiling_on_sc=False),
        out_type=(jax.ShapeDtypeStruct((_EP,), jnp.float32),
                  jax.ShapeDtypeStruct((_EP,), jnp.float32),
                  jax.ShapeDtypeStruct((_NC, 2, _N), jnp.float32)),
        scratch_types=[
            pltpu.VMEM((_CHUNK,), jnp.int32),
            pltpu.VMEM((_CHUNK,), jnp.int32),
            pltpu.VMEM((_CHUNK,), jnp.float32),
            pltpu.VMEM((_CHUNK,), jnp.float32),
            pltpu.VMEM((_CHUNK,), jnp.float32),
            pltpu.VMEM((_CHUNK,), jnp.float32),
            pltpu.VMEM((_CHUNK,), jnp.float32),
            pltpu.VMEM((_CHUNK,), jnp.float32),
            pltpu.VMEM((_CHUNK,), jnp.float32),
            pltpu.VMEM((_CHUNK,), jnp.float32),
            pltpu.VMEM((32,), jnp.float32),
            pltpu.VMEM_SHARED((_N,), jnp.float32),
            pltpu.VMEM_SHARED((_N,), jnp.float32),
            pltpu.SemaphoreType.DMA,
        ],
    )
    def body(src, dst, ae0, ae1, as0, as1, ad0, ad1, mvec, zn,
             p0, p1, dsum,
             isrc, idst, g_as0, g_as1, g_ad0, g_ad1, e0, e1, pb0, pb1,
             mv, d0, d1, sem):
        sid = lax.axis_index("s")
        cid = lax.axis_index("c")
        wid = sid * _NC + cid

        @pl.when(sid == 0)
        def _():
            pltpu.sync_copy(zn, d0)
            pltpu.sync_copy(zn, d1)
        plsc.subcore_barrier()

        pltpu.sync_copy(mvec, mv)

        def chunk(t, carry):
            base = wid * _PER_W + t * _CHUNK
            pltpu.sync_copy(src.at[pl.ds(base, _CHUNK)], isrc)
            pltpu.sync_copy(dst.at[pl.ds(base, _CHUNK)], idst)
            pltpu.sync_copy(ae0.at[pl.ds(base, _CHUNK)], e0)
            pltpu.sync_copy(ae1.at[pl.ds(base, _CHUNK)], e1)
            c1 = pltpu.async_copy(as0.at[isrc], g_as0, sem)
            c2 = pltpu.async_copy(as1.at[isrc], g_as1, sem)
            c3 = pltpu.async_copy(ad0.at[idst], g_ad0, sem)
            c4 = pltpu.async_copy(ad1.at[idst], g_ad1, sem)
            c1.wait(); c2.wait(); c3.wait(); c4.wait()
            m0 = mv[pl.ds(0, 16)]
            m1 = mv[pl.ds(16, 16)]
            for j in range(_CHUNK // 16):
                sl = pl.ds(16 * j, 16)
                a0 = g_as0[sl] + g_ad0[sl] + e0[sl]
                a0 = jnp.where(a0 > 0.0, a0, 0.2 * a0)
                pb0[sl] = jnp.exp(a0 - m0)
                a1 = g_as1[sl] + g_ad1[sl] + e1[sl]
                a1 = jnp.where(a1 > 0.0, a1, 0.2 * a1)
                pb1[sl] = jnp.exp(a1 - m1)
            pltpu.sync_copy(pb0, d0.at[idst], add=True)
            pltpu.sync_copy(pb1, d1.at[idst], add=True)
            pltpu.sync_copy(pb0, p0.at[pl.ds(base, _CHUNK)])
            pltpu.sync_copy(pb1, p1.at[pl.ds(base, _CHUNK)])
            return carry

        lax.fori_loop(0, _NCHUNKS, chunk, 0)
        plsc.subcore_barrier()

        @pl.when(sid == 0)
        def _():
            pltpu.sync_copy(d0, dsum.at[cid, 0])
            pltpu.sync_copy(d1, dsum.at[cid, 1])

    return body


def _msg_kernel(nslices):
    mesh = plsc.VectorSubcoreMesh(core_axis_name="c", subcore_axis_name="s")
    half = nslices // 2

    @functools.partial(
        pl.kernel, mesh=mesh,
        compiler_params=pltpu.CompilerParams(use_tc_tiling_on_sc=False),
        out_type=jax.ShapeDtypeStruct((_NC, nslices, _N, 32), jnp.float32),
        scratch_types=[
            pltpu.VMEM((_CHUNK,), jnp.int32),
            pltpu.VMEM((_CHUNK,), jnp.int32),
            pltpu.VMEM((_CHUNK, 32), jnp.float32),
            pltpu.VMEM((_CHUNK,), jnp.float32),
            pltpu.VMEM_SHARED((_N, 32), jnp.float32),
            pltpu.SemaphoreType.DMA,
        ],
    )
    def body(*args):
        (src, dst, p0, p1) = args[:4]
        xls = args[4:4 + nslices]
        z32 = args[4 + nslices]
        out = args[5 + nslices]
        isrc, idst, rows, pbuf, acc, sem = args[6 + nslices:]
        sid = lax.axis_index("s")
        cid = lax.axis_index("c")
        wid = sid * _NC + cid
        ps = (p0, p1)

        for s in range(nslices):
            @pl.when(sid == 0)
            def _():
                pltpu.sync_copy(z32, acc)
            plsc.subcore_barrier()

            def chunk(t, carry):
                base = wid * _PER_W + t * _CHUNK
                pltpu.sync_copy(src.at[pl.ds(base, _CHUNK)], isrc)
                pltpu.sync_copy(dst.at[pl.ds(base, _CHUNK)], idst)
                pltpu.sync_copy(ps[s // half].at[pl.ds(base, _CHUNK)], pbuf)
                pltpu.async_copy(xls[s].at[isrc], rows, sem).wait()

                def scale(g, c2):
                    pg = pbuf[pl.ds(16 * g, 16)]
                    for l in range(16):
                        e = 16 * g + l
                        pv = jnp.broadcast_to(pg[l], (16,))
                        rows[e, pl.ds(0, 16)] = rows[e, pl.ds(0, 16)] * pv
                        rows[e, pl.ds(16, 16)] = rows[e, pl.ds(16, 16)] * pv
                    return c2

                lax.fori_loop(0, _CHUNK // 16, scale, 0)
                pltpu.sync_copy(rows, acc.at[idst], add=True)
                return carry

            lax.fori_loop(0, _NCHUNKS, chunk, 0)
            plsc.subcore_barrier()

            @pl.when(sid == 0)
            def _():
                pltpu.sync_copy(acc, out.at[cid, s])
            plsc.subcore_barrier()

    return body


def kernel(x, edge_index, edge_attr, params):
    src = jnp.pad(edge_index[0].astype(jnp.int32), (0, _EP - _E))
    dst = jnp.pad(edge_index[1].astype(jnp.int32), (0, _EP - _E))

    n_pad = 50176
    e_pad = 800768

    we_cols = []
    for p in params:
        h_, c_ = p['att_e'].shape
        we_vec = (p['W_e'].reshape(-1, h_, c_) * p['att_e'][None]).sum(-1)
        we_cols.append(we_vec)
    we_cat = jnp.concatenate(we_cols, axis=1)
    ae_all = _mm(_pad_to(edge_attr, e_pad, 128),
                 _pad_to(we_cat, 128, 128))[:_E, :16]
    ae_max = jnp.max(ae_all, axis=0)
    ae_pad_t = jnp.pad(ae_all.T, ((0, 0), (0, _EP - _E)),
                       constant_values=-1e9)

    zn = jnp.zeros((_N,), jnp.float32)
    z32 = jnp.zeros((_N, 32), jnp.float32)

    alpha_fn = _alpha_kernel()
    msg4_fn = _msg_kernel(4)
    msg2_fn = _msg_kernel(2)

    h = _pad_to(x, n_pad, 128)
    num_layers = len(params)
    for i, p in enumerate(params):
        heads, c = p['att_src'].shape
        hc = heads * c
        in_d = p['W'].shape[0]
        w_src = (p['W'].reshape(in_d, heads, c) * p['att_src'][None]).sum(-1)
        w_dst = (p['W'].reshape(in_d, heads, c) * p['att_dst'][None]).sum(-1)
        w_cat = jnp.concatenate([p['W'], w_src, w_dst], axis=1)
        m_cols = 256 if hc + 2 * heads > 128 else 128
        out = _mm(h, _pad_to(w_cat, 128, m_cols))
        xl = out[:_N, :hc]
        a_src = out[:_N, hc:hc + heads]
        a_dst = out[:_N, hc + heads:hc + 2 * heads]

        mh = (jnp.max(a_src, axis=0) + jnp.max(a_dst, axis=0)
              + ae_max[2 * i:2 * i + 2])
        mh = jnp.maximum(mh, 0.0)
        mvec = jnp.repeat(mh, 16)

        p0, p1, dsum = alpha_fn(
            src, dst, ae_pad_t[2 * i], ae_pad_t[2 * i + 1],
            a_src[:, 0], a_src[:, 1], a_dst[:, 0], a_dst[:, 1], mvec, zn)
        denom = dsum[0] + dsum[1]

        nsl = 4 if hc == 128 else 2
        if nsl == 4:
            xls = xl.reshape(_N, 4, 32).transpose(1, 0, 2)
        else:
            xls = jnp.pad(xl.reshape(_N, heads, c),
                          ((0, 0), (0, 0), (0, 32 - c))).transpose(1, 0, 2)
        msg_fn = msg4_fn if nsl == 4 else msg2_fn
        accp = msg_fn(src, dst, p0, p1, *(xls[s] for s in range(nsl)), z32)
        acc = accp[0] + accp[1]

        if i < num_layers - 1:
            accn = acc / (denom[jnp.repeat(jnp.arange(2), 2)][:, :, None]
                          + 1e-16)
            hn = accn.transpose(1, 0, 2).reshape(_N, hc) + p['b']
            hn = jnp.maximum(hn, 0.0)
            h = jnp.pad(hn, ((0, n_pad - _N), (0, 0)))
        else:
            accn = acc[:, :, :3] / (denom[:, :, None] + 1e-16)
            return accn.mean(axis=0) + p['b']


# speedup vs baseline: 28.1702x; 1.1596x over previous
import functools

import jax
import jax.numpy as jnp
from jax import lax
from jax.experimental import pallas as pl
from jax.experimental.pallas import tpu as pltpu
from jax.experimental.pallas import tpu_sc as plsc

_N = 50000
_E = 800000
_EP = 802816
_CHUNK = 128
_NCHUNKS = 196
_PER_W = _CHUNK * _NCHUNKS
_NC = 2
_NS = 16


def _mm_body(x_ref, w_ref, o_ref):
    o_ref[...] = jnp.dot(x_ref[...], w_ref[...],
                         preferred_element_type=jnp.float32)


def _mm(x, w, bn=1024):
    n, k = x.shape
    m = w.shape[1]
    return pl.pallas_call(
        _mm_body,
        grid=(n // bn,),
        in_specs=[pl.BlockSpec((bn, k), lambda i: (i, 0)),
                  pl.BlockSpec((k, m), lambda i: (0, 0))],
        out_specs=pl.BlockSpec((bn, m), lambda i: (i, 0)),
        out_shape=jax.ShapeDtypeStruct((n, m), jnp.float32),
    )(x, w)


def _pad_to(a, rows, cols):
    return jnp.pad(a, ((0, rows - a.shape[0]), (0, cols - a.shape[1])))


def _worker_id():
    return lax.axis_index("s") * _NC + lax.axis_index("c")


def _alpha_kernel():
    mesh = plsc.VectorSubcoreMesh(core_axis_name="c", subcore_axis_name="s")

    @functools.partial(
        pl.kernel, mesh=mesh,
        compiler_params=pltpu.CompilerParams(use_tc_tiling_on_sc=False),
        out_type=(jax.ShapeDtypeStruct((_EP,), jnp.float32),
                  jax.ShapeDtypeStruct((_EP,), jnp.float32),
                  jax.ShapeDtypeStruct((_NC, 2, _N), jnp.float32)),
        scratch_types=[
            pltpu.VMEM((_CHUNK,), jnp.int32),
            pltpu.VMEM((_CHUNK,), jnp.int32),
            pltpu.VMEM((_CHUNK,), jnp.float32),
            pltpu.VMEM((_CHUNK,), jnp.float32),
            pltpu.VMEM((_CHUNK,), jnp.float32),
            pltpu.VMEM((_CHUNK,), jnp.float32),
            pltpu.VMEM((_CHUNK,), jnp.float32),
            pltpu.VMEM((_CHUNK,), jnp.float32),
            pltpu.VMEM((_CHUNK,), jnp.float32),
            pltpu.VMEM((_CHUNK,), jnp.float32),
            pltpu.VMEM((32,), jnp.float32),
            pltpu.VMEM_SHARED((_N,), jnp.float32),
            pltpu.VMEM_SHARED((_N,), jnp.float32),
            pltpu.SemaphoreType.DMA,
        ],
    )
    def body(src, dst, ae0, ae1, as0, as1, ad0, ad1, mvec, zn,
             p0, p1, dsum,
             isrc, idst, g_as0, g_as1, g_ad0, g_ad1, e0, e1, pb0, pb1,
             mv, d0, d1, sem):
        sid = lax.axis_index("s")
        cid = lax.axis_index("c")
        wid = sid * _NC + cid

        @pl.when(sid == 0)
        def _():
            pltpu.sync_copy(zn, d0)
            pltpu.sync_copy(zn, d1)
        plsc.subcore_barrier()

        pltpu.sync_copy(mvec, mv)

        def chunk(t, carry):
            base = wid * _PER_W + t * _CHUNK
            pltpu.sync_copy(src.at[pl.ds(base, _CHUNK)], isrc)
            pltpu.sync_copy(dst.at[pl.ds(base, _CHUNK)], idst)
            pltpu.sync_copy(ae0.at[pl.ds(base, _CHUNK)], e0)
            pltpu.sync_copy(ae1.at[pl.ds(base, _CHUNK)], e1)
            c1 = pltpu.async_copy(as0.at[isrc], g_as0, sem)
            c2 = pltpu.async_copy(as1.at[isrc], g_as1, sem)
            c3 = pltpu.async_copy(ad0.at[idst], g_ad0, sem)
            c4 = pltpu.async_copy(ad1.at[idst], g_ad1, sem)
            c1.wait(); c2.wait(); c3.wait(); c4.wait()
            m0 = mv[pl.ds(0, 16)]
            m1 = mv[pl.ds(16, 16)]
            for j in range(_CHUNK // 16):
                sl = pl.ds(16 * j, 16)
                a0 = g_as0[sl] + g_ad0[sl] + e0[sl]
                a0 = jnp.where(a0 > 0.0, a0, 0.2 * a0)
                pb0[sl] = jnp.exp(a0 - m0)
                a1 = g_as1[sl] + g_ad1[sl] + e1[sl]
                a1 = jnp.where(a1 > 0.0, a1, 0.2 * a1)
                pb1[sl] = jnp.exp(a1 - m1)
            pltpu.sync_copy(pb0, d0.at[idst], add=True)
            pltpu.sync_copy(pb1, d1.at[idst], add=True)
            pltpu.sync_copy(pb0, p0.at[pl.ds(base, _CHUNK)])
            pltpu.sync_copy(pb1, p1.at[pl.ds(base, _CHUNK)])
            return carry

        lax.fori_loop(0, _NCHUNKS, chunk, 0)
        plsc.subcore_barrier()

        @pl.when(sid == 0)
        def _():
            pltpu.sync_copy(d0, dsum.at[cid, 0])
            pltpu.sync_copy(d1, dsum.at[cid, 1])

    return body


def _msg_kernel(nslices):
    mesh = plsc.VectorSubcoreMesh(core_axis_name="c", subcore_axis_name="s")
    half = nslices // 2

    @functools.partial(
        pl.kernel, mesh=mesh,
        compiler_params=pltpu.CompilerParams(use_tc_tiling_on_sc=False),
        out_type=jax.ShapeDtypeStruct((_NC, nslices, _N, 32), jnp.float32),
        scratch_types=[
            pltpu.VMEM((_CHUNK,), jnp.int32),
            pltpu.VMEM((_CHUNK,), jnp.int32),
            pltpu.VMEM((_CHUNK,), jnp.int32),
            pltpu.VMEM((_CHUNK,), jnp.int32),
            pltpu.VMEM((_CHUNK, 32), jnp.float32),
            pltpu.VMEM((_CHUNK, 32), jnp.float32),
            pltpu.VMEM((_CHUNK,), jnp.float32),
            pltpu.VMEM((_CHUNK,), jnp.float32),
            pltpu.VMEM_SHARED((_N, 32), jnp.float32),
            pltpu.SemaphoreType.DMA,
            pltpu.SemaphoreType.DMA,
        ],
    )
    def body(*args):
        (src, dst, p0, p1) = args[:4]
        xls = args[4:4 + nslices]
        z32 = args[4 + nslices]
        out = args[5 + nslices]
        (isrc0, isrc1, idst0, idst1, rows0, rows1, pb_0, pb_1,
         acc, sem0, sem1) = args[6 + nslices:]
        sid = lax.axis_index("s")
        cid = lax.axis_index("c")
        wid = sid * _NC + cid
        ps = (p0, p1)
        isrc_b = (isrc0, isrc1)
        idst_b = (idst0, idst1)
        rows_b = (rows0, rows1)
        pb_b = (pb_0, pb_1)
        sem_b = (sem0, sem1)
        stripe = _N // _NS

        for s in range(nslices):
            ph = ps[s // half]
            xl_s = xls[s]

            def stage(b, t):
                base = wid * _PER_W + t * _CHUNK
                pltpu.sync_copy(src.at[pl.ds(base, _CHUNK)], isrc_b[b])
                pltpu.sync_copy(dst.at[pl.ds(base, _CHUNK)], idst_b[b])
                pltpu.sync_copy(ph.at[pl.ds(base, _CHUNK)], pb_b[b])
                pltpu.async_copy(xl_s.at[isrc_b[b]], rows_b[b], sem_b[b])

            def process(b):
                pltpu.make_async_copy(
                    xl_s.at[pl.ds(0, _CHUNK)], rows_b[b], sem_b[b]).wait()
                rows = rows_b[b]
                pbuf = pb_b[b]

                def scale(g, c2):
                    pg = pbuf[pl.ds(16 * g, 16)]
                    for l in range(16):
                        e = 16 * g + l
                        pv = jnp.broadcast_to(pg[l], (16,))
                        rows[e, pl.ds(0, 16)] = rows[e, pl.ds(0, 16)] * pv
                        rows[e, pl.ds(16, 16)] = rows[e, pl.ds(16, 16)] * pv
                    return c2

                lax.fori_loop(0, _CHUNK // 16, scale, 0)
                pltpu.sync_copy(rows, acc.at[idst_b[b]], add=True)

            row0 = sid * stripe
            pltpu.sync_copy(z32.at[pl.ds(row0, stripe)],
                            acc.at[pl.ds(row0, stripe)])
            plsc.subcore_barrier()

            stage(0, 0)
            stage(1, 1)

            def outer(g, carry):
                t0 = 2 * g
                process(0)
                stage(0, jnp.minimum(t0 + 2, _NCHUNKS - 1))
                process(1)
                stage(1, jnp.minimum(t0 + 3, _NCHUNKS - 1))
                return carry

            lax.fori_loop(0, _NCHUNKS // 2, outer, 0)
            for b in range(2):
                pltpu.make_async_copy(
                    xl_s.at[pl.ds(0, _CHUNK)], rows_b[b], sem_b[b]).wait()
            plsc.subcore_barrier()
            pltpu.sync_copy(acc.at[pl.ds(row0, stripe)],
                            out.at[cid, s, pl.ds(row0, stripe)])
            plsc.subcore_barrier()

    return body


def kernel(x, edge_index, edge_attr, params):
    src = jnp.pad(edge_index[0].astype(jnp.int32), (0, _EP - _E))
    dst = jnp.pad(edge_index[1].astype(jnp.int32), (0, _EP - _E))

    n_pad = 50176
    e_pad = 800768

    we_cols = []
    for p in params:
        h_, c_ = p['att_e'].shape
        we_vec = (p['W_e'].reshape(-1, h_, c_) * p['att_e'][None]).sum(-1)
        we_cols.append(we_vec)
    we_cat = jnp.concatenate(we_cols, axis=1)
    ae_all = _mm(_pad_to(edge_attr, e_pad, 128),
                 _pad_to(we_cat, 128, 128))[:_E, :16]
    ae_max = jnp.max(ae_all, axis=0)
    ae_pad_t = jnp.pad(ae_all.T, ((0, 0), (0, _EP - _E)),
                       constant_values=-1e9)

    zn = jnp.zeros((_N,), jnp.float32)
    z32 = jnp.zeros((_N, 32), jnp.float32)

    alpha_fn = _alpha_kernel()
    msg4_fn = _msg_kernel(4)
    msg2_fn = _msg_kernel(2)

    h = _pad_to(x, n_pad, 128)
    num_layers = len(params)
    for i, p in enumerate(params):
        heads, c = p['att_src'].shape
        hc = heads * c
        in_d = p['W'].shape[0]
        w_src = (p['W'].reshape(in_d, heads, c) * p['att_src'][None]).sum(-1)
        w_dst = (p['W'].reshape(in_d, heads, c) * p['att_dst'][None]).sum(-1)
        w_cat = jnp.concatenate([p['W'], w_src, w_dst], axis=1)
        m_cols = 256 if hc + 2 * heads > 128 else 128
        out = _mm(h, _pad_to(w_cat, 128, m_cols))
        xl = out[:_N, :hc]
        a_src = out[:_N, hc:hc + heads]
        a_dst = out[:_N, hc + heads:hc + 2 * heads]

        mh = (jnp.max(a_src, axis=0) + jnp.max(a_dst, axis=0)
              + ae_max[2 * i:2 * i + 2])
        mh = jnp.maximum(mh, 0.0)
        mvec = jnp.repeat(mh, 16)

        p0, p1, dsum = alpha_fn(
            src, dst, ae_pad_t[2 * i], ae_pad_t[2 * i + 1],
            a_src[:, 0], a_src[:, 1], a_dst[:, 0], a_dst[:, 1], mvec, zn)
        denom = dsum[0] + dsum[1]

        nsl = 4 if hc == 128 else 2
        if nsl == 4:
            xls = xl.reshape(_N, 4, 32).transpose(1, 0, 2)
        else:
            xls = jnp.pad(xl.reshape(_N, heads, c),
                          ((0, 0), (0, 0), (0, 32 - c))).transpose(1, 0, 2)
        msg_fn = msg4_fn if nsl == 4 else msg2_fn
        accp = msg_fn(src, dst, p0, p1, *(xls[s] for s in range(nsl)), z32)
        acc = accp[0] + accp[1]

        if i < num_layers - 1:
            accn = acc / (denom[jnp.repeat(jnp.arange(2), 2)][:, :, None]
                          + 1e-16)
            hn = accn.transpose(1, 0, 2).reshape(_N, hc) + p['b']
            hn = jnp.maximum(hn, 0.0)
            h = jnp.pad(hn, ((0, n_pad - _N), (0, 0)))
        else:
            accn = acc[:, :, :3] / (denom[:, :, None] + 1e-16)
            return accn.mean(axis=0) + p['b']
